# baseline (speedup 1.0000x reference)
import functools

import jax
import jax.numpy as jnp
from jax import lax
from jax.experimental import pallas as pl
from jax.experimental.pallas import tpu as pltpu
from jax.experimental.pallas import tpu_sc as plsc

N = 10000
E = 160000
F_IN = 256
HID = 256
CLS = 64

K = 128
NCHUNK = E // K
NSUB = 16
ROWS_PER_SUB = 624
ZROWS = 104
MBLK = 1000


def _zero_acc(s, zbuf, acc, sem):
    def zrow(r, carry):
        for j in range(128 // 16):
            zbuf[r, pl.ds(j * 16, 16)] = jnp.zeros((16,), jnp.float32)
        return carry
    lax.fori_loop(0, K, zrow, 0)
    r0 = s * ROWS_PER_SUB
    tail = ROWS_PER_SUB % K
    for t in range(ROWS_PER_SUB // K):
        pltpu.async_copy(zbuf, acc.at[pl.ds(r0 + t * K, K)], sem)
    pltpu.async_copy(zbuf.at[pl.ds(0, tail)],
                     acc.at[pl.ds(r0 + 4 * K, tail)], sem)

    @pl.when(s == NSUB - 1)
    def _():
        pltpu.async_copy(zbuf.at[pl.ds(0, 16)],
                         acc.at[pl.ds(NSUB * ROWS_PER_SUB, 16)], sem)

    for t in range(ROWS_PER_SUB // K):
        pltpu.make_async_copy(zbuf, acc.at[pl.ds(r0 + t * K, K)], sem).wait()
    pltpu.make_async_copy(zbuf.at[pl.ds(0, tail)],
                          acc.at[pl.ds(r0 + 4 * K, tail)], sem).wait()

    @pl.when(s == NSUB - 1)
    def _():
        pltpu.make_async_copy(zbuf.at[pl.ds(0, 16)],
                              acc.at[pl.ds(NSUB * ROWS_PER_SUB, 16)],
                              sem).wait()


def _writeback(s, acc, out_ref, sem):
    r0 = s * ROWS_PER_SUB
    tail = ROWS_PER_SUB % K
    for t in range(ROWS_PER_SUB // K):
        sl = pl.ds(r0 + t * K, K)
        pltpu.async_copy(acc.at[sl], out_ref.at[sl], sem)
    sl4 = pl.ds(r0 + 4 * K, tail)
    pltpu.async_copy(acc.at[sl4], out_ref.at[sl4], sem)

    @pl.when(s == NSUB - 1)
    def _():
        sl = pl.ds(NSUB * ROWS_PER_SUB, 16)
        pltpu.async_copy(acc.at[sl], out_ref.at[sl], sem)

    for t in range(ROWS_PER_SUB // K):
        sl = pl.ds(r0 + t * K, K)
        pltpu.make_async_copy(acc.at[sl], out_ref.at[sl], sem).wait()
    pltpu.make_async_copy(acc.at[sl4], out_ref.at[sl4], sem).wait()

    @pl.when(s == NSUB - 1)
    def _():
        sl = pl.ds(NSUB * ROWS_PER_SUB, 16)
        pltpu.make_async_copy(acc.at[sl], out_ref.at[sl], sem).wait()


def _pipeline(cnt, gbase, h_ref, src2d, dst2d, acc, rows, sg, ss, sidx, didx, si):
    def idx_load(j):
        slot = lax.rem(j, 4)
        pltpu.async_copy(src2d.at[gbase + j], sidx.at[slot], si)
        pltpu.async_copy(dst2d.at[gbase + j], didx.at[slot], si)

    def wait_idx(j):
        slot = lax.rem(j, 4)
        pltpu.make_async_copy(src2d.at[gbase + j], sidx.at[slot], si).wait()
        pltpu.make_async_copy(dst2d.at[gbase + j], didx.at[slot], si).wait()

    def gather(j, b):
        pltpu.async_copy(h_ref.at[sidx.at[lax.rem(j, 4)]], rows[b], sg[b])

    def wait_gather(j, b):
        pltpu.make_async_copy(
            h_ref.at[sidx.at[lax.rem(j, 4)]], rows[b], sg[b]).wait()

    def scatter(j, b):
        pltpu.async_copy(rows[b], acc.at[didx.at[lax.rem(j, 4)]],
                         ss[b], add=True)

    def wait_scatter(j, b):
        pltpu.make_async_copy(
            rows[b], acc.at[didx.at[lax.rem(j, 4)]], ss[b]).wait()

    def stage(j, b):
        m = (b + 2) % 3
        wait_gather(j, b)
        scatter(j, b)

        @pl.when(j + 2 < cnt)
        def _():
            @pl.when(j >= 1)
            def _():
                wait_scatter(j - 1, m)
            wait_idx(j + 2)
            gather(j + 2, m)

        @pl.when(j + 3 < cnt)
        def _():
            idx_load(j + 3)


    def body(j, carry):
        for r in range(3):
            @pl.when(lax.rem(j, 3) == r)
            def _():
                stage(j, r)
        return carry
    lax.fori_loop(0, cnt, body, 0)

    for r in range(3):
        @pl.when(lax.rem(cnt, 3) == r)
        def _():
            wait_scatter(cnt - 3, r)
            wait_scatter(cnt - 2, (r + 1) % 3)
            wait_scatter(cnt - 1, (r + 2) % 3)


def _prologue_idx(gbase, src2d, dst2d, sidx, didx, si):
    for j in (0, 1):
        pltpu.async_copy(src2d.at[gbase + j], sidx.at[j], si)
        pltpu.async_copy(dst2d.at[gbase + j], didx.at[j], si)


def _prologue_gather(gbase, h_ref, src2d, dst2d, rows, sg, sidx, didx, si):
    for j in (0, 1):
        pltpu.make_async_copy(src2d.at[gbase + j], sidx.at[j], si).wait()
        pltpu.make_async_copy(dst2d.at[gbase + j], didx.at[j], si).wait()
    pltpu.async_copy(h_ref.at[sidx.at[0]], rows[0], sg[0])
    pltpu.async_copy(h_ref.at[sidx.at[1]], rows[1], sg[1])
    pltpu.async_copy(src2d.at[gbase + 2], sidx.at[2], si)
    pltpu.async_copy(dst2d.at[gbase + 2], didx.at[2], si)


_SPMM_SCRATCH = [
    pltpu.VMEM((4, K), jnp.int32),
    pltpu.VMEM((4, K), jnp.int32),
    pltpu.VMEM((K, 128), jnp.float32),
    pltpu.VMEM((K, 128), jnp.float32),
    pltpu.VMEM((K, 128), jnp.float32),
    pltpu.VMEM_SHARED((N, 128), jnp.float32),
] + [pltpu.SemaphoreType.DMA] * 7


def _make_spmm_col_split():
    mesh = plsc.VectorSubcoreMesh(core_axis_name="c", subcore_axis_name="s")
    base_cnt = NCHUNK // NSUB
    extra = NCHUNK % NSUB

    @functools.partial(
        pl.kernel,
        out_type=(jax.ShapeDtypeStruct((N, 128), jnp.float32),
                  jax.ShapeDtypeStruct((N, 128), jnp.float32)),
        mesh=mesh,
        scratch_types=_SPMM_SCRATCH,
    )
    def spmm(h0, h1, src2d, dst2d, out0, out1,
             sidx, didx, rows0, rows1, rows2, acc,
             sg0, sg1, sg2, ss0, ss1, ss2, si):
        c = lax.axis_index("c")
        s = lax.axis_index("s")
        cnt = jnp.where(s < extra, base_cnt + 1, base_cnt)
        gbase = s * base_cnt + jnp.minimum(s, extra)
        rows = (rows0, rows2, rows1)
        sg = (sg0, sg1, sg2)
        ss = (ss0, ss1, ss2)
        _prologue_idx(gbase, src2d, dst2d, sidx, didx, si)
        _zero_acc(s, rows1, acc, ss2)

        @pl.when(c == 0)
        def _():
            _prologue_gather(gbase, h0, src2d, dst2d, rows, sg,
                             sidx, didx, si)

        @pl.when(c == 1)
        def _():
            _prologue_gather(gbase, h1, src2d, dst2d, rows, sg,
                             sidx, didx, si)
        plsc.subcore_barrier()

        @pl.when(c == 0)
        def _():
            _pipeline(cnt, gbase, h0, src2d, dst2d, acc,
                      rows, sg, ss, sidx, didx, si)

        @pl.when(c == 1)
        def _():
            _pipeline(cnt, gbase, h1, src2d, dst2d, acc,
                      rows, sg, ss, sidx, didx, si)

        plsc.subcore_barrier()

        @pl.when(c == 0)
        def _():
            _writeback(s, acc, out0, si)

        @pl.when(c == 1)
        def _():
            _writeback(s, acc, out1, si)

    return spmm


def _make_spmm_edge_split():
    mesh = plsc.VectorSubcoreMesh(core_axis_name="c", subcore_axis_name="s")
    nw = 2 * NSUB
    base_cnt = NCHUNK // nw
    extra = NCHUNK % nw

    @functools.partial(
        pl.kernel,
        out_type=(jax.ShapeDtypeStruct((N, 128), jnp.float32),
                  jax.ShapeDtypeStruct((N, 128), jnp.float32)),
        mesh=mesh,
        scratch_types=_SPMM_SCRATCH,
    )
    def spmm(h, src2d, dst2d, out0, out1,
             sidx, didx, rows0, rows1, rows2, acc,
             sg0, sg1, sg2, ss0, ss1, ss2, si):
        c = lax.axis_index("c")
        s = lax.axis_index("s")
        w = c * NSUB + s
        cnt = jnp.where(w < extra, base_cnt + 1, base_cnt)
        gbase = w * base_cnt + jnp.minimum(w, extra)
        rows = (rows0, rows2, rows1)
        sg = (sg0, sg1, sg2)
        ss = (ss0, ss1, ss2)
        _prologue_idx(gbase, src2d, dst2d, sidx, didx, si)
        _zero_acc(s, rows1, acc, ss2)
        _prologue_gather(gbase, h, src2d, dst2d, rows, sg, sidx, didx, si)
        plsc.subcore_barrier()

        _pipeline(cnt, gbase, h, src2d, dst2d, acc,
                  rows, sg, ss, sidx, didx, si)

        plsc.subcore_barrier()

        @pl.when(c == 0)
        def _():
            _writeback(s, acc, out0, si)

        @pl.when(c == 1)
        def _():
            _writeback(s, acc, out1, si)

    return spmm


_spmm128 = _make_spmm_col_split()
_spmm_last = _make_spmm_edge_split()


def _gemm0(x, w):
    def body(x_ref, w_ref, oa, ob):
        h = jnp.dot(x_ref[...], w_ref[...], preferred_element_type=jnp.float32)
        oa[...] = h[:, :128]
        ob[...] = h[:, 128:]
    return pl.pallas_call(
        body,
        grid=(N // MBLK,),
        in_specs=[pl.BlockSpec((MBLK, F_IN), lambda i: (i, 0)),
                  pl.BlockSpec((F_IN, HID), lambda i: (0, 0))],
        out_specs=[pl.BlockSpec((MBLK, 128), lambda i: (i, 0))] * 2,
        out_shape=[jax.ShapeDtypeStruct((N, 128), jnp.float32)] * 2,
    )(x, w)


def _gemm_relu(ha, hb, w, dout):
    dh = dout // 2

    def body(a_ref, b_ref, w_ref, oa, ob):
        xa = jnp.maximum(a_ref[...], 0.0)
        xb = jnp.maximum(b_ref[...], 0.0)
        h = (jnp.dot(xa, w_ref[:128, :], preferred_element_type=jnp.float32)
             + jnp.dot(xb, w_ref[128:, :], preferred_element_type=jnp.float32))
        oa[...] = h[:, :dh]
        ob[...] = h[:, dh:]

    return pl.pallas_call(
        body,
        grid=(N // MBLK,),
        in_specs=[pl.BlockSpec((MBLK, 128), lambda i: (i, 0)),
                  pl.BlockSpec((MBLK, 128), lambda i: (i, 0)),
                  pl.BlockSpec((HID, dout), lambda i: (0, 0))],
        out_specs=[pl.BlockSpec((MBLK, dh), lambda i: (i, 0))] * 2,
        out_shape=[jax.ShapeDtypeStruct((N, dh), jnp.float32)] * 2,
    )(ha, hb, w)


def _gemm_relu_wide(ha, hb, w):
    def body(a_ref, b_ref, w_ref, o_ref):
        xa = jnp.maximum(a_ref[...], 0.0)
        xb = jnp.maximum(b_ref[...], 0.0)
        o_ref[...] = (
            jnp.dot(xa, w_ref[:128, :], preferred_element_type=jnp.float32)
            + jnp.dot(xb, w_ref[128:, :], preferred_element_type=jnp.float32))

    return pl.pallas_call(
        body,
        grid=(N // MBLK,),
        in_specs=[pl.BlockSpec((MBLK, 128), lambda i: (i, 0)),
                  pl.BlockSpec((MBLK, 128), lambda i: (i, 0)),
                  pl.BlockSpec((HID, 128), lambda i: (0, 0))],
        out_specs=pl.BlockSpec((MBLK, 128), lambda i: (i, 0)),
        out_shape=jax.ShapeDtypeStruct((N, 128), jnp.float32),
    )(ha, hb, w)


def _log_softmax_sum(p0, p1):
    def body(a_ref, b_ref, o_ref):
        x = (a_ref[...] + b_ref[...])[:, :CLS]
        m = jnp.max(x, axis=1, keepdims=True)
        sh = x - m
        o_ref[...] = sh - jnp.log(jnp.sum(jnp.exp(sh), axis=1, keepdims=True))

    return pl.pallas_call(
        body,
        grid=(N // MBLK,),
        in_specs=[pl.BlockSpec((MBLK, 128), lambda i: (i, 0)),
                  pl.BlockSpec((MBLK, 128), lambda i: (i, 0))],
        out_specs=pl.BlockSpec((MBLK, CLS), lambda i: (i, 0)),
        out_shape=jax.ShapeDtypeStruct((N, CLS), jnp.float32),
    )(p0, p1)


def kernel(inputs, edge_index, W0, W1, W2, epoch):
    src2d = edge_index[0].reshape(NCHUNK, K)
    dst2d = edge_index[1].reshape(NCHUNK, K)
    w2p = jnp.pad(W2, ((0, 0), (0, 128 - CLS)))

    h0a, h0b = _gemm0(inputs, W0)
    a0a, a0b = _spmm128(h0a, h0b, src2d, dst2d)
    h1a, h1b = _gemm_relu(a0a, a0b, W1, HID)
    a1a, a1b = _spmm128(h1a, h1b, src2d, dst2d)
    h2 = _gemm_relu_wide(a1a, a1b, w2p)
    p0, p1 = _spmm_last(h2, src2d, dst2d)
    return _log_softmax_sum(p0, p1)


# speedup vs baseline: 1.0346x; 1.0346x over previous
import functools

import jax
import jax.numpy as jnp
from jax import lax
from jax.experimental import pallas as pl
from jax.experimental.pallas import tpu as pltpu
from jax.experimental.pallas import tpu_sc as plsc

N = 10000
E = 160000
F_IN = 256
HID = 256
CLS = 64

K = 128
NCHUNK = E // K
NSUB = 16
ROWS_PER_SUB = 624
ZROWS = 104
MBLK = 2000


def _zero_acc(s, zbuf, acc, sem):
    def zrow(r, carry):
        for j in range(128 // 16):
            zbuf[r, pl.ds(j * 16, 16)] = jnp.zeros((16,), jnp.float32)
        return carry
    lax.fori_loop(0, K, zrow, 0)
    r0 = s * ROWS_PER_SUB
    tail = ROWS_PER_SUB % K
    for t in range(ROWS_PER_SUB // K):
        pltpu.async_copy(zbuf, acc.at[pl.ds(r0 + t * K, K)], sem)
    pltpu.async_copy(zbuf.at[pl.ds(0, tail)],
                     acc.at[pl.ds(r0 + 4 * K, tail)], sem)

    @pl.when(s == NSUB - 1)
    def _():
        pltpu.async_copy(zbuf.at[pl.ds(0, 16)],
                         acc.at[pl.ds(NSUB * ROWS_PER_SUB, 16)], sem)

    for t in range(ROWS_PER_SUB // K):
        pltpu.make_async_copy(zbuf, acc.at[pl.ds(r0 + t * K, K)], sem).wait()
    pltpu.make_async_copy(zbuf.at[pl.ds(0, tail)],
                          acc.at[pl.ds(r0 + 4 * K, tail)], sem).wait()

    @pl.when(s == NSUB - 1)
    def _():
        pltpu.make_async_copy(zbuf.at[pl.ds(0, 16)],
                              acc.at[pl.ds(NSUB * ROWS_PER_SUB, 16)],
                              sem).wait()


def _writeback(s, acc, out_ref, sem):
    r0 = s * ROWS_PER_SUB
    tail = ROWS_PER_SUB % K
    for t in range(ROWS_PER_SUB // K):
        sl = pl.ds(r0 + t * K, K)
        pltpu.async_copy(acc.at[sl], out_ref.at[sl], sem)
    sl4 = pl.ds(r0 + 4 * K, tail)
    pltpu.async_copy(acc.at[sl4], out_ref.at[sl4], sem)

    @pl.when(s == NSUB - 1)
    def _():
        sl = pl.ds(NSUB * ROWS_PER_SUB, 16)
        pltpu.async_copy(acc.at[sl], out_ref.at[sl], sem)

    for t in range(ROWS_PER_SUB // K):
        sl = pl.ds(r0 + t * K, K)
        pltpu.make_async_copy(acc.at[sl], out_ref.at[sl], sem).wait()
    pltpu.make_async_copy(acc.at[sl4], out_ref.at[sl4], sem).wait()

    @pl.when(s == NSUB - 1)
    def _():
        sl = pl.ds(NSUB * ROWS_PER_SUB, 16)
        pltpu.make_async_copy(acc.at[sl], out_ref.at[sl], sem).wait()


def _pipeline(cnt, gbase, h_ref, src2d, dst2d, acc, rows, sg, ss, sidx, didx, si):
    def idx_load(j):
        slot = lax.rem(j, 4)
        pltpu.async_copy(src2d.at[gbase + j], sidx.at[slot], si)
        pltpu.async_copy(dst2d.at[gbase + j], didx.at[slot], si)

    def wait_idx(j):
        slot = lax.rem(j, 4)
        pltpu.make_async_copy(src2d.at[gbase + j], sidx.at[slot], si).wait()
        pltpu.make_async_copy(dst2d.at[gbase + j], didx.at[slot], si).wait()

    def gather(j, b):
        pltpu.async_copy(h_ref.at[sidx.at[lax.rem(j, 4)]], rows[b], sg[b])

    def wait_gather(j, b):
        pltpu.make_async_copy(
            h_ref.at[sidx.at[lax.rem(j, 4)]], rows[b], sg[b]).wait()

    def scatter(j, b):
        pltpu.async_copy(rows[b], acc.at[didx.at[lax.rem(j, 4)]],
                         ss[b], add=True)

    def wait_scatter(j, b):
        pltpu.make_async_copy(
            rows[b], acc.at[didx.at[lax.rem(j, 4)]], ss[b]).wait()

    def stage(j, b):
        m = (b + 2) % 3
        wait_gather(j, b)
        scatter(j, b)

        @pl.when(j + 2 < cnt)
        def _():
            @pl.when(j >= 1)
            def _():
                wait_scatter(j - 1, m)
            wait_idx(j + 2)
            gather(j + 2, m)

        @pl.when(j + 3 < cnt)
        def _():
            idx_load(j + 3)


    def body(j, carry):
        for r in range(3):
            @pl.when(lax.rem(j, 3) == r)
            def _():
                stage(j, r)
        return carry
    lax.fori_loop(0, cnt, body, 0)

    for r in range(3):
        @pl.when(lax.rem(cnt, 3) == r)
        def _():
            wait_scatter(cnt - 3, r)
            wait_scatter(cnt - 2, (r + 1) % 3)
            wait_scatter(cnt - 1, (r + 2) % 3)


def _prologue_idx(gbase, src2d, dst2d, sidx, didx, si):
    for j in (0, 1):
        pltpu.async_copy(src2d.at[gbase + j], sidx.at[j], si)
        pltpu.async_copy(dst2d.at[gbase + j], didx.at[j], si)


def _prologue_gather(gbase, h_ref, src2d, dst2d, rows, sg, sidx, didx, si):
    for j in (0, 1):
        pltpu.make_async_copy(src2d.at[gbase + j], sidx.at[j], si).wait()
        pltpu.make_async_copy(dst2d.at[gbase + j], didx.at[j], si).wait()
    pltpu.async_copy(h_ref.at[sidx.at[0]], rows[0], sg[0])
    pltpu.async_copy(h_ref.at[sidx.at[1]], rows[1], sg[1])
    pltpu.async_copy(src2d.at[gbase + 2], sidx.at[2], si)
    pltpu.async_copy(dst2d.at[gbase + 2], didx.at[2], si)


_SPMM_SCRATCH = [
    pltpu.VMEM((4, K), jnp.int32),
    pltpu.VMEM((4, K), jnp.int32),
    pltpu.VMEM((K, 128), jnp.float32),
    pltpu.VMEM((K, 128), jnp.float32),
    pltpu.VMEM((K, 128), jnp.float32),
    pltpu.VMEM_SHARED((N, 128), jnp.float32),
] + [pltpu.SemaphoreType.DMA] * 7


def _make_spmm_col_split():
    mesh = plsc.VectorSubcoreMesh(core_axis_name="c", subcore_axis_name="s")
    base_cnt = NCHUNK // NSUB
    extra = NCHUNK % NSUB

    @functools.partial(
        pl.kernel,
        out_type=(jax.ShapeDtypeStruct((N, 128), jnp.float32),
                  jax.ShapeDtypeStruct((N, 128), jnp.float32)),
        mesh=mesh,
        scratch_types=_SPMM_SCRATCH,
    )
    def spmm(h0, h1, src2d, dst2d, out0, out1,
             sidx, didx, rows0, rows1, rows2, acc,
             sg0, sg1, sg2, ss0, ss1, ss2, si):
        c = lax.axis_index("c")
        s = lax.axis_index("s")
        cnt = jnp.where(s < extra, base_cnt + 1, base_cnt)
        gbase = s * base_cnt + jnp.minimum(s, extra)
        rows = (rows0, rows2, rows1)
        sg = (sg0, sg1, sg2)
        ss = (ss0, ss1, ss2)
        _prologue_idx(gbase, src2d, dst2d, sidx, didx, si)
        _zero_acc(s, rows1, acc, ss2)

        @pl.when(c == 0)
        def _():
            _prologue_gather(gbase, h0, src2d, dst2d, rows, sg,
                             sidx, didx, si)

        @pl.when(c == 1)
        def _():
            _prologue_gather(gbase, h1, src2d, dst2d, rows, sg,
                             sidx, didx, si)
        plsc.subcore_barrier()

        @pl.when(c == 0)
        def _():
            _pipeline(cnt, gbase, h0, src2d, dst2d, acc,
                      rows, sg, ss, sidx, didx, si)

        @pl.when(c == 1)
        def _():
            _pipeline(cnt, gbase, h1, src2d, dst2d, acc,
                      rows, sg, ss, sidx, didx, si)

        plsc.subcore_barrier()

        @pl.when(c == 0)
        def _():
            _writeback(s, acc, out0, si)

        @pl.when(c == 1)
        def _():
            _writeback(s, acc, out1, si)

    return spmm


def _make_spmm_edge_split():
    mesh = plsc.VectorSubcoreMesh(core_axis_name="c", subcore_axis_name="s")
    nw = 2 * NSUB
    base_cnt = NCHUNK // nw
    extra = NCHUNK % nw

    @functools.partial(
        pl.kernel,
        out_type=(jax.ShapeDtypeStruct((N, 128), jnp.float32),
                  jax.ShapeDtypeStruct((N, 128), jnp.float32)),
        mesh=mesh,
        scratch_types=_SPMM_SCRATCH,
    )
    def spmm(h, src2d, dst2d, out0, out1,
             sidx, didx, rows0, rows1, rows2, acc,
             sg0, sg1, sg2, ss0, ss1, ss2, si):
        c = lax.axis_index("c")
        s = lax.axis_index("s")
        w = c * NSUB + s
        cnt = jnp.where(w < extra, base_cnt + 1, base_cnt)
        gbase = w * base_cnt + jnp.minimum(w, extra)
        rows = (rows0, rows2, rows1)
        sg = (sg0, sg1, sg2)
        ss = (ss0, ss1, ss2)
        _prologue_idx(gbase, src2d, dst2d, sidx, didx, si)
        _zero_acc(s, rows1, acc, ss2)
        _prologue_gather(gbase, h, src2d, dst2d, rows, sg, sidx, didx, si)
        plsc.subcore_barrier()

        _pipeline(cnt, gbase, h, src2d, dst2d, acc,
                  rows, sg, ss, sidx, didx, si)

        plsc.subcore_barrier()

        @pl.when(c == 0)
        def _():
            _writeback(s, acc, out0, si)

        @pl.when(c == 1)
        def _():
            _writeback(s, acc, out1, si)

    return spmm


_spmm128 = _make_spmm_col_split()
_spmm_last = _make_spmm_edge_split()


def _gemm0(x, w):
    def body(x_ref, w_ref, oa, ob):
        h = jnp.dot(x_ref[...], w_ref[...], preferred_element_type=jnp.float32)
        oa[...] = h[:, :128]
        ob[...] = h[:, 128:]
    return pl.pallas_call(
        body,
        grid=(N // MBLK,),
        in_specs=[pl.BlockSpec((MBLK, F_IN), lambda i: (i, 0)),
                  pl.BlockSpec((F_IN, HID), lambda i: (0, 0))],
        out_specs=[pl.BlockSpec((MBLK, 128), lambda i: (i, 0))] * 2,
        out_shape=[jax.ShapeDtypeStruct((N, 128), jnp.float32)] * 2,
    )(x, w)


def _gemm_relu(ha, hb, w, dout):
    dh = dout // 2

    def body(a_ref, b_ref, w_ref, oa, ob):
        xa = jnp.maximum(a_ref[...], 0.0)
        xb = jnp.maximum(b_ref[...], 0.0)
        h = (jnp.dot(xa, w_ref[:128, :], preferred_element_type=jnp.float32)
             + jnp.dot(xb, w_ref[128:, :], preferred_element_type=jnp.float32))
        oa[...] = h[:, :dh]
        ob[...] = h[:, dh:]

    return pl.pallas_call(
        body,
        grid=(N // MBLK,),
        in_specs=[pl.BlockSpec((MBLK, 128), lambda i: (i, 0)),
                  pl.BlockSpec((MBLK, 128), lambda i: (i, 0)),
                  pl.BlockSpec((HID, dout), lambda i: (0, 0))],
        out_specs=[pl.BlockSpec((MBLK, dh), lambda i: (i, 0))] * 2,
        out_shape=[jax.ShapeDtypeStruct((N, dh), jnp.float32)] * 2,
    )(ha, hb, w)


def _gemm_relu_wide(ha, hb, w):
    def body(a_ref, b_ref, w_ref, o_ref):
        xa = jnp.maximum(a_ref[...], 0.0)
        xb = jnp.maximum(b_ref[...], 0.0)
        o_ref[...] = (
            jnp.dot(xa, w_ref[:128, :], preferred_element_type=jnp.float32)
            + jnp.dot(xb, w_ref[128:, :], preferred_element_type=jnp.float32))

    return pl.pallas_call(
        body,
        grid=(N // MBLK,),
        in_specs=[pl.BlockSpec((MBLK, 128), lambda i: (i, 0)),
                  pl.BlockSpec((MBLK, 128), lambda i: (i, 0)),
                  pl.BlockSpec((HID, 128), lambda i: (0, 0))],
        out_specs=pl.BlockSpec((MBLK, 128), lambda i: (i, 0)),
        out_shape=jax.ShapeDtypeStruct((N, 128), jnp.float32),
    )(ha, hb, w)


def _log_softmax_sum(p0, p1):
    def body(a_ref, b_ref, o_ref):
        x = (a_ref[...] + b_ref[...])[:, :CLS]
        m = jnp.max(x, axis=1, keepdims=True)
        sh = x - m
        o_ref[...] = sh - jnp.log(jnp.sum(jnp.exp(sh), axis=1, keepdims=True))

    return pl.pallas_call(
        body,
        grid=(N // MBLK,),
        in_specs=[pl.BlockSpec((MBLK, 128), lambda i: (i, 0)),
                  pl.BlockSpec((MBLK, 128), lambda i: (i, 0))],
        out_specs=pl.BlockSpec((MBLK, CLS), lambda i: (i, 0)),
        out_shape=jax.ShapeDtypeStruct((N, CLS), jnp.float32),
    )(p0, p1)


def kernel(inputs, edge_index, W0, W1, W2, epoch):
    src2d = edge_index[0].reshape(NCHUNK, K)
    dst2d = edge_index[1].reshape(NCHUNK, K)
    w2p = jnp.pad(W2, ((0, 0), (0, 128 - CLS)))

    h0a, h0b = _gemm0(inputs, W0)
    a0a, a0b = _spmm128(h0a, h0b, src2d, dst2d)
    h1a, h1b = _gemm_relu(a0a, a0b, W1, HID)
    a1a, a1b = _spmm128(h1a, h1b, src2d, dst2d)
    h2 = _gemm_relu_wide(a1a, a1b, w2p)
    p0, p1 = _spmm_last(h2, src2d, dst2d)
    return _log_softmax_sum(p0, p1)


# speedup vs baseline: 1.0689x; 1.0331x over previous
import functools

import jax
import jax.numpy as jnp
from jax import lax
from jax.experimental import pallas as pl
from jax.experimental.pallas import tpu as pltpu
from jax.experimental.pallas import tpu_sc as plsc

N = 10000
E = 160000
F_IN = 256
HID = 256
CLS = 64

K = 128
NCHUNK = E // K
NSUB = 16
ROWS_PER_SUB = 624
ZROWS = 104
MBLK = 2000


def _zero_acc(s, zbuf, acc, sem):
    def zrow(r, carry):
        for j in range(128 // 16):
            zbuf[r, pl.ds(j * 16, 16)] = jnp.zeros((16,), jnp.float32)
        return carry
    lax.fori_loop(0, K, zrow, 0)
    r0 = s * ROWS_PER_SUB
    tail = ROWS_PER_SUB % K
    for t in range(ROWS_PER_SUB // K):
        pltpu.async_copy(zbuf, acc.at[pl.ds(r0 + t * K, K)], sem)
    pltpu.async_copy(zbuf.at[pl.ds(0, tail)],
                     acc.at[pl.ds(r0 + 4 * K, tail)], sem)

    @pl.when(s == NSUB - 1)
    def _():
        pltpu.async_copy(zbuf.at[pl.ds(0, 16)],
                         acc.at[pl.ds(NSUB * ROWS_PER_SUB, 16)], sem)

    for t in range(ROWS_PER_SUB // K):
        pltpu.make_async_copy(zbuf, acc.at[pl.ds(r0 + t * K, K)], sem).wait()
    pltpu.make_async_copy(zbuf.at[pl.ds(0, tail)],
                          acc.at[pl.ds(r0 + 4 * K, tail)], sem).wait()

    @pl.when(s == NSUB - 1)
    def _():
        pltpu.make_async_copy(zbuf.at[pl.ds(0, 16)],
                              acc.at[pl.ds(NSUB * ROWS_PER_SUB, 16)],
                              sem).wait()


def _writeback(s, acc, out_ref, sem):
    r0 = s * ROWS_PER_SUB
    tail = ROWS_PER_SUB % K
    for t in range(ROWS_PER_SUB // K):
        sl = pl.ds(r0 + t * K, K)
        pltpu.async_copy(acc.at[sl], out_ref.at[sl], sem)
    sl4 = pl.ds(r0 + 4 * K, tail)
    pltpu.async_copy(acc.at[sl4], out_ref.at[sl4], sem)

    @pl.when(s == NSUB - 1)
    def _():
        sl = pl.ds(NSUB * ROWS_PER_SUB, 16)
        pltpu.async_copy(acc.at[sl], out_ref.at[sl], sem)

    for t in range(ROWS_PER_SUB // K):
        sl = pl.ds(r0 + t * K, K)
        pltpu.make_async_copy(acc.at[sl], out_ref.at[sl], sem).wait()
    pltpu.make_async_copy(acc.at[sl4], out_ref.at[sl4], sem).wait()

    @pl.when(s == NSUB - 1)
    def _():
        sl = pl.ds(NSUB * ROWS_PER_SUB, 16)
        pltpu.make_async_copy(acc.at[sl], out_ref.at[sl], sem).wait()


def _pipeline(cnt, gbase, h_ref, edges3d, acc, rows, sg, ss, ibuf, si):
    def idx_load(j):
        pltpu.async_copy(edges3d.at[gbase + j], ibuf.at[lax.rem(j, 4)], si)

    def wait_idx(j):
        pltpu.make_async_copy(
            edges3d.at[gbase + j], ibuf.at[lax.rem(j, 4)], si).wait()

    def gather(j, b):
        pltpu.async_copy(h_ref.at[ibuf.at[lax.rem(j, 4), 0]], rows[b], sg[b])

    def wait_gather(j, b):
        pltpu.make_async_copy(
            h_ref.at[ibuf.at[lax.rem(j, 4), 0]], rows[b], sg[b]).wait()

    def scatter(j, b):
        pltpu.async_copy(rows[b], acc.at[ibuf.at[lax.rem(j, 4), 1]],
                         ss[b], add=True)

    def wait_scatter(j, b):
        pltpu.make_async_copy(
            rows[b], acc.at[ibuf.at[lax.rem(j, 4), 1]], ss[b]).wait()

    def stage(j, b):
        m = (b + 2) % 3
        wait_gather(j, b)
        scatter(j, b)

        @pl.when(j + 2 < cnt)
        def _():
            @pl.when(j >= 1)
            def _():
                wait_scatter(j - 1, m)
            wait_idx(j + 2)
            gather(j + 2, m)

        @pl.when(j + 3 < cnt)
        def _():
            idx_load(j + 3)


    def body(j, carry):
        for r in range(3):
            @pl.when(lax.rem(j, 3) == r)
            def _():
                stage(j, r)
        return carry
    lax.fori_loop(0, cnt, body, 0)

    for r in range(3):
        @pl.when(lax.rem(cnt, 3) == r)
        def _():
            wait_scatter(cnt - 3, r)
            wait_scatter(cnt - 2, (r + 1) % 3)
            wait_scatter(cnt - 1, (r + 2) % 3)


def _prologue_idx(gbase, edges3d, ibuf, si):
    for j in (0, 1):
        pltpu.async_copy(edges3d.at[gbase + j], ibuf.at[j], si)


def _prologue_gather(gbase, h_ref, edges3d, rows, sg, ibuf, si):
    for j in (0, 1):
        pltpu.make_async_copy(edges3d.at[gbase + j], ibuf.at[j], si).wait()
    pltpu.async_copy(h_ref.at[ibuf.at[0, 0]], rows[0], sg[0])
    pltpu.async_copy(h_ref.at[ibuf.at[1, 0]], rows[1], sg[1])
    pltpu.async_copy(edges3d.at[gbase + 2], ibuf.at[2], si)


_SPMM_SCRATCH = [
    pltpu.VMEM((4, 2, K), jnp.int32),
    pltpu.VMEM((K, 128), jnp.float32),
    pltpu.VMEM((K, 128), jnp.float32),
    pltpu.VMEM((K, 128), jnp.float32),
    pltpu.VMEM_SHARED((N, 128), jnp.float32),
] + [pltpu.SemaphoreType.DMA] * 7


def _make_spmm_col_split():
    mesh = plsc.VectorSubcoreMesh(core_axis_name="c", subcore_axis_name="s")
    base_cnt = NCHUNK // NSUB
    extra = NCHUNK % NSUB

    @functools.partial(
        pl.kernel,
        out_type=(jax.ShapeDtypeStruct((N, 128), jnp.float32),
                  jax.ShapeDtypeStruct((N, 128), jnp.float32)),
        mesh=mesh,
        scratch_types=_SPMM_SCRATCH,
    )
    def spmm(h0, h1, edges3d, out0, out1,
             ibuf, rows0, rows1, rows2, acc,
             sg0, sg1, sg2, ss0, ss1, ss2, si):
        c = lax.axis_index("c")
        s = lax.axis_index("s")
        cnt = jnp.where(s < extra, base_cnt + 1, base_cnt)
        gbase = s * base_cnt + jnp.minimum(s, extra)
        rows = (rows0, rows2, rows1)
        sg = (sg0, sg1, sg2)
        ss = (ss0, ss1, ss2)
        _prologue_idx(gbase, edges3d, ibuf, si)
        _zero_acc(s, rows1, acc, ss2)

        @pl.when(c == 0)
        def _():
            _prologue_gather(gbase, h0, edges3d, rows, sg, ibuf, si)

        @pl.when(c == 1)
        def _():
            _prologue_gather(gbase, h1, edges3d, rows, sg, ibuf, si)
        plsc.subcore_barrier()

        @pl.when(c == 0)
        def _():
            _pipeline(cnt, gbase, h0, edges3d, acc, rows, sg, ss, ibuf, si)

        @pl.when(c == 1)
        def _():
            _pipeline(cnt, gbase, h1, edges3d, acc, rows, sg, ss, ibuf, si)

        plsc.subcore_barrier()

        @pl.when(c == 0)
        def _():
            _writeback(s, acc, out0, si)

        @pl.when(c == 1)
        def _():
            _writeback(s, acc, out1, si)

    return spmm


def _make_spmm_edge_split():
    mesh = plsc.VectorSubcoreMesh(core_axis_name="c", subcore_axis_name="s")
    nw = 2 * NSUB
    base_cnt = NCHUNK // nw
    extra = NCHUNK % nw

    @functools.partial(
        pl.kernel,
        out_type=(jax.ShapeDtypeStruct((N, 128), jnp.float32),
                  jax.ShapeDtypeStruct((N, 128), jnp.float32)),
        mesh=mesh,
        scratch_types=_SPMM_SCRATCH,
    )
    def spmm(h, edges3d, out0, out1,
             ibuf, rows0, rows1, rows2, acc,
             sg0, sg1, sg2, ss0, ss1, ss2, si):
        c = lax.axis_index("c")
        s = lax.axis_index("s")
        w = c * NSUB + s
        cnt = jnp.where(w < extra, base_cnt + 1, base_cnt)
        gbase = w * base_cnt + jnp.minimum(w, extra)
        rows = (rows0, rows2, rows1)
        sg = (sg0, sg1, sg2)
        ss = (ss0, ss1, ss2)
        _prologue_idx(gbase, edges3d, ibuf, si)
        _zero_acc(s, rows1, acc, ss2)
        _prologue_gather(gbase, h, edges3d, rows, sg, ibuf, si)
        plsc.subcore_barrier()

        _pipeline(cnt, gbase, h, edges3d, acc, rows, sg, ss, ibuf, si)

        plsc.subcore_barrier()

        @pl.when(c == 0)
        def _():
            _writeback(s, acc, out0, si)

        @pl.when(c == 1)
        def _():
            _writeback(s, acc, out1, si)

    return spmm


_spmm128 = _make_spmm_col_split()
_spmm_last = _make_spmm_edge_split()


def _gemm0(x, w):
    def body(x_ref, w_ref, oa, ob):
        h = jnp.dot(x_ref[...], w_ref[...], preferred_element_type=jnp.float32)
        oa[...] = h[:, :128]
        ob[...] = h[:, 128:]
    return pl.pallas_call(
        body,
        grid=(N // MBLK,),
        in_specs=[pl.BlockSpec((MBLK, F_IN), lambda i: (i, 0)),
                  pl.BlockSpec((F_IN, HID), lambda i: (0, 0))],
        out_specs=[pl.BlockSpec((MBLK, 128), lambda i: (i, 0))] * 2,
        out_shape=[jax.ShapeDtypeStruct((N, 128), jnp.float32)] * 2,
    )(x, w)


def _gemm_relu(ha, hb, w, dout):
    dh = dout // 2

    def body(a_ref, b_ref, w_ref, oa, ob):
        xa = jnp.maximum(a_ref[...], 0.0)
        xb = jnp.maximum(b_ref[...], 0.0)
        h = (jnp.dot(xa, w_ref[:128, :], preferred_element_type=jnp.float32)
             + jnp.dot(xb, w_ref[128:, :], preferred_element_type=jnp.float32))
        oa[...] = h[:, :dh]
        ob[...] = h[:, dh:]

    return pl.pallas_call(
        body,
        grid=(N // MBLK,),
        in_specs=[pl.BlockSpec((MBLK, 128), lambda i: (i, 0)),
                  pl.BlockSpec((MBLK, 128), lambda i: (i, 0)),
                  pl.BlockSpec((HID, dout), lambda i: (0, 0))],
        out_specs=[pl.BlockSpec((MBLK, dh), lambda i: (i, 0))] * 2,
        out_shape=[jax.ShapeDtypeStruct((N, dh), jnp.float32)] * 2,
    )(ha, hb, w)


def _gemm_relu_wide(ha, hb, w):
    def body(a_ref, b_ref, w_ref, o_ref):
        xa = jnp.maximum(a_ref[...], 0.0)
        xb = jnp.maximum(b_ref[...], 0.0)
        o_ref[...] = (
            jnp.dot(xa, w_ref[:128, :], preferred_element_type=jnp.float32)
            + jnp.dot(xb, w_ref[128:, :], preferred_element_type=jnp.float32))

    return pl.pallas_call(
        body,
        grid=(N // MBLK,),
        in_specs=[pl.BlockSpec((MBLK, 128), lambda i: (i, 0)),
                  pl.BlockSpec((MBLK, 128), lambda i: (i, 0)),
                  pl.BlockSpec((HID, 128), lambda i: (0, 0))],
        out_specs=pl.BlockSpec((MBLK, 128), lambda i: (i, 0)),
        out_shape=jax.ShapeDtypeStruct((N, 128), jnp.float32),
    )(ha, hb, w)


def _log_softmax_sum(p0, p1):
    def body(a_ref, b_ref, o_ref):
        x = (a_ref[...] + b_ref[...])[:, :CLS]
        m = jnp.max(x, axis=1, keepdims=True)
        sh = x - m
        o_ref[...] = sh - jnp.log(jnp.sum(jnp.exp(sh), axis=1, keepdims=True))

    return pl.pallas_call(
        body,
        grid=(N // MBLK,),
        in_specs=[pl.BlockSpec((MBLK, 128), lambda i: (i, 0)),
                  pl.BlockSpec((MBLK, 128), lambda i: (i, 0))],
        out_specs=pl.BlockSpec((MBLK, CLS), lambda i: (i, 0)),
        out_shape=jax.ShapeDtypeStruct((N, CLS), jnp.float32),
    )(p0, p1)


def kernel(inputs, edge_index, W0, W1, W2, epoch):
    edges3d = edge_index.reshape(2, NCHUNK, K).transpose(1, 0, 2)
    w2p = jnp.pad(W2, ((0, 0), (0, 128 - CLS)))

    h0a, h0b = _gemm0(inputs, W0)
    a0a, a0b = _spmm128(h0a, h0b, edges3d)
    h1a, h1b = _gemm_relu(a0a, a0b, W1, HID)
    a1a, a1b = _spmm128(h1a, h1b, edges3d)
    h2 = _gemm_relu_wide(a1a, a1b, w2p)
    p0, p1 = _spmm_last(h2, edges3d)
    return _log_softmax_sum(p0, p1)


# speedup vs baseline: 1.0952x; 1.0246x over previous
import functools

import jax
import jax.numpy as jnp
from jax import lax
from jax.experimental import pallas as pl
from jax.experimental.pallas import tpu as pltpu
from jax.experimental.pallas import tpu_sc as plsc

N = 10000
E = 160000
F_IN = 256
HID = 256
CLS = 64

K = 128
NCHUNK = E // K
NSUB = 16
ROWS_PER_SUB = 624
ZROWS = 104
MBLK = 5000


def _zero_acc(s, zbuf, acc, sem):
    def zrow(r, carry):
        for j in range(128 // 16):
            zbuf[r, pl.ds(j * 16, 16)] = jnp.zeros((16,), jnp.float32)
        return carry
    lax.fori_loop(0, K, zrow, 0)
    r0 = s * ROWS_PER_SUB
    tail = ROWS_PER_SUB % K
    for t in range(ROWS_PER_SUB // K):
        pltpu.async_copy(zbuf, acc.at[pl.ds(r0 + t * K, K)], sem)
    pltpu.async_copy(zbuf.at[pl.ds(0, tail)],
                     acc.at[pl.ds(r0 + 4 * K, tail)], sem)

    @pl.when(s == NSUB - 1)
    def _():
        pltpu.async_copy(zbuf.at[pl.ds(0, 16)],
                         acc.at[pl.ds(NSUB * ROWS_PER_SUB, 16)], sem)

    for t in range(ROWS_PER_SUB // K):
        pltpu.make_async_copy(zbuf, acc.at[pl.ds(r0 + t * K, K)], sem).wait()
    pltpu.make_async_copy(zbuf.at[pl.ds(0, tail)],
                          acc.at[pl.ds(r0 + 4 * K, tail)], sem).wait()

    @pl.when(s == NSUB - 1)
    def _():
        pltpu.make_async_copy(zbuf.at[pl.ds(0, 16)],
                              acc.at[pl.ds(NSUB * ROWS_PER_SUB, 16)],
                              sem).wait()


def _writeback(s, acc, out_ref, sem):
    r0 = s * ROWS_PER_SUB
    tail = ROWS_PER_SUB % K
    for t in range(ROWS_PER_SUB // K):
        sl = pl.ds(r0 + t * K, K)
        pltpu.async_copy(acc.at[sl], out_ref.at[sl], sem)
    sl4 = pl.ds(r0 + 4 * K, tail)
    pltpu.async_copy(acc.at[sl4], out_ref.at[sl4], sem)

    @pl.when(s == NSUB - 1)
    def _():
        sl = pl.ds(NSUB * ROWS_PER_SUB, 16)
        pltpu.async_copy(acc.at[sl], out_ref.at[sl], sem)

    for t in range(ROWS_PER_SUB // K):
        sl = pl.ds(r0 + t * K, K)
        pltpu.make_async_copy(acc.at[sl], out_ref.at[sl], sem).wait()
    pltpu.make_async_copy(acc.at[sl4], out_ref.at[sl4], sem).wait()

    @pl.when(s == NSUB - 1)
    def _():
        sl = pl.ds(NSUB * ROWS_PER_SUB, 16)
        pltpu.make_async_copy(acc.at[sl], out_ref.at[sl], sem).wait()


def _pipeline(cnt, gbase, h_ref, edges3d, acc, rows, sg, ss, ibuf, si):
    def idx_load(j):
        pltpu.async_copy(edges3d.at[gbase + j], ibuf.at[lax.rem(j, 4)], si)

    def wait_idx(j):
        pltpu.make_async_copy(
            edges3d.at[gbase + j], ibuf.at[lax.rem(j, 4)], si).wait()

    def gather(j, b):
        pltpu.async_copy(h_ref.at[ibuf.at[lax.rem(j, 4), 0]], rows[b], sg[b])

    def wait_gather(j, b):
        pltpu.make_async_copy(
            h_ref.at[ibuf.at[lax.rem(j, 4), 0]], rows[b], sg[b]).wait()

    def scatter(j, b):
        pltpu.async_copy(rows[b], acc.at[ibuf.at[lax.rem(j, 4), 1]],
                         ss[b], add=True)

    def wait_scatter(j, b):
        pltpu.make_async_copy(
            rows[b], acc.at[ibuf.at[lax.rem(j, 4), 1]], ss[b]).wait()

    def stage(j, b):
        m = (b + 2) % 3
        wait_gather(j, b)
        scatter(j, b)

        @pl.when(j + 2 < cnt)
        def _():
            @pl.when(j >= 1)
            def _():
                wait_scatter(j - 1, m)
            wait_idx(j + 2)
            gather(j + 2, m)

        @pl.when(j + 3 < cnt)
        def _():
            idx_load(j + 3)


    def body(j, carry):
        for r in range(3):
            @pl.when(lax.rem(j, 3) == r)
            def _():
                stage(j, r)
        return carry
    lax.fori_loop(0, cnt, body, 0)

    for r in range(3):
        @pl.when(lax.rem(cnt, 3) == r)
        def _():
            wait_scatter(cnt - 3, r)
            wait_scatter(cnt - 2, (r + 1) % 3)
            wait_scatter(cnt - 1, (r + 2) % 3)


def _prologue_idx(gbase, edges3d, ibuf, si):
    for j in (0, 1):
        pltpu.async_copy(edges3d.at[gbase + j], ibuf.at[j], si)


def _prologue_gather(gbase, h_ref, edges3d, rows, sg, ibuf, si):
    for j in (0, 1):
        pltpu.make_async_copy(edges3d.at[gbase + j], ibuf.at[j], si).wait()
    pltpu.async_copy(h_ref.at[ibuf.at[0, 0]], rows[0], sg[0])
    pltpu.async_copy(h_ref.at[ibuf.at[1, 0]], rows[1], sg[1])
    pltpu.async_copy(edges3d.at[gbase + 2], ibuf.at[2], si)


_SPMM_SCRATCH = [
    pltpu.VMEM((4, 2, K), jnp.int32),
    pltpu.VMEM((K, 128), jnp.float32),
    pltpu.VMEM((K, 128), jnp.float32),
    pltpu.VMEM((K, 128), jnp.float32),
    pltpu.VMEM_SHARED((N, 128), jnp.float32),
] + [pltpu.SemaphoreType.DMA] * 7


def _make_spmm_col_split():
    mesh = plsc.VectorSubcoreMesh(core_axis_name="c", subcore_axis_name="s")
    base_cnt = NCHUNK // NSUB
    extra = NCHUNK % NSUB

    @functools.partial(
        pl.kernel,
        out_type=(jax.ShapeDtypeStruct((N, 128), jnp.float32),
                  jax.ShapeDtypeStruct((N, 128), jnp.float32)),
        mesh=mesh,
        scratch_types=_SPMM_SCRATCH,
    )
    def spmm(h0, h1, edges3d, out0, out1,
             ibuf, rows0, rows1, rows2, acc,
             sg0, sg1, sg2, ss0, ss1, ss2, si):
        c = lax.axis_index("c")
        s = lax.axis_index("s")
        cnt = jnp.where(s < extra, base_cnt + 1, base_cnt)
        gbase = s * base_cnt + jnp.minimum(s, extra)
        rows = (rows0, rows2, rows1)
        sg = (sg0, sg1, sg2)
        ss = (ss0, ss1, ss2)
        _prologue_idx(gbase, edges3d, ibuf, si)
        _zero_acc(s, rows1, acc, ss2)

        @pl.when(c == 0)
        def _():
            _prologue_gather(gbase, h0, edges3d, rows, sg, ibuf, si)

        @pl.when(c == 1)
        def _():
            _prologue_gather(gbase, h1, edges3d, rows, sg, ibuf, si)
        plsc.subcore_barrier()

        @pl.when(c == 0)
        def _():
            _pipeline(cnt, gbase, h0, edges3d, acc, rows, sg, ss, ibuf, si)

        @pl.when(c == 1)
        def _():
            _pipeline(cnt, gbase, h1, edges3d, acc, rows, sg, ss, ibuf, si)

        plsc.subcore_barrier()

        @pl.when(c == 0)
        def _():
            _writeback(s, acc, out0, si)

        @pl.when(c == 1)
        def _():
            _writeback(s, acc, out1, si)

    return spmm


def _make_spmm_edge_split():
    mesh = plsc.VectorSubcoreMesh(core_axis_name="c", subcore_axis_name="s")
    nw = 2 * NSUB
    base_cnt = NCHUNK // nw
    extra = NCHUNK % nw

    @functools.partial(
        pl.kernel,
        out_type=(jax.ShapeDtypeStruct((N, 128), jnp.float32),
                  jax.ShapeDtypeStruct((N, 128), jnp.float32)),
        mesh=mesh,
        scratch_types=_SPMM_SCRATCH,
    )
    def spmm(h, edges3d, out0, out1,
             ibuf, rows0, rows1, rows2, acc,
             sg0, sg1, sg2, ss0, ss1, ss2, si):
        c = lax.axis_index("c")
        s = lax.axis_index("s")
        w = c * NSUB + s
        cnt = jnp.where(w < extra, base_cnt + 1, base_cnt)
        gbase = w * base_cnt + jnp.minimum(w, extra)
        rows = (rows0, rows2, rows1)
        sg = (sg0, sg1, sg2)
        ss = (ss0, ss1, ss2)
        _prologue_idx(gbase, edges3d, ibuf, si)
        _zero_acc(s, rows1, acc, ss2)
        _prologue_gather(gbase, h, edges3d, rows, sg, ibuf, si)
        plsc.subcore_barrier()

        _pipeline(cnt, gbase, h, edges3d, acc, rows, sg, ss, ibuf, si)

        plsc.subcore_barrier()

        @pl.when(c == 0)
        def _():
            _writeback(s, acc, out0, si)

        @pl.when(c == 1)
        def _():
            _writeback(s, acc, out1, si)

    return spmm


_spmm128 = _make_spmm_col_split()
_spmm_last = _make_spmm_edge_split()


def _gemm0(x, w):
    def body(x_ref, w_ref, oa, ob):
        h = jnp.dot(x_ref[...], w_ref[...], preferred_element_type=jnp.float32)
        oa[...] = h[:, :128]
        ob[...] = h[:, 128:]
    return pl.pallas_call(
        body,
        grid=(N // MBLK,),
        in_specs=[pl.BlockSpec((MBLK, F_IN), lambda i: (i, 0)),
                  pl.BlockSpec((F_IN, HID), lambda i: (0, 0))],
        out_specs=[pl.BlockSpec((MBLK, 128), lambda i: (i, 0))] * 2,
        out_shape=[jax.ShapeDtypeStruct((N, 128), jnp.float32)] * 2,
    )(x, w)


def _gemm_relu(ha, hb, w, dout):
    dh = dout // 2

    def body(a_ref, b_ref, w_ref, oa, ob):
        xa = jnp.maximum(a_ref[...], 0.0)
        xb = jnp.maximum(b_ref[...], 0.0)
        h = (jnp.dot(xa, w_ref[:128, :], preferred_element_type=jnp.float32)
             + jnp.dot(xb, w_ref[128:, :], preferred_element_type=jnp.float32))
        oa[...] = h[:, :dh]
        ob[...] = h[:, dh:]

    return pl.pallas_call(
        body,
        grid=(N // MBLK,),
        in_specs=[pl.BlockSpec((MBLK, 128), lambda i: (i, 0)),
                  pl.BlockSpec((MBLK, 128), lambda i: (i, 0)),
                  pl.BlockSpec((HID, dout), lambda i: (0, 0))],
        out_specs=[pl.BlockSpec((MBLK, dh), lambda i: (i, 0))] * 2,
        out_shape=[jax.ShapeDtypeStruct((N, dh), jnp.float32)] * 2,
    )(ha, hb, w)


def _gemm_relu_wide(ha, hb, w):
    def body(a_ref, b_ref, w_ref, o_ref):
        xa = jnp.maximum(a_ref[...], 0.0)
        xb = jnp.maximum(b_ref[...], 0.0)
        o_ref[...] = (
            jnp.dot(xa, w_ref[:128, :], preferred_element_type=jnp.float32)
            + jnp.dot(xb, w_ref[128:, :], preferred_element_type=jnp.float32))

    return pl.pallas_call(
        body,
        grid=(N // MBLK,),
        in_specs=[pl.BlockSpec((MBLK, 128), lambda i: (i, 0)),
                  pl.BlockSpec((MBLK, 128), lambda i: (i, 0)),
                  pl.BlockSpec((HID, 128), lambda i: (0, 0))],
        out_specs=pl.BlockSpec((MBLK, 128), lambda i: (i, 0)),
        out_shape=jax.ShapeDtypeStruct((N, 128), jnp.float32),
    )(ha, hb, w)


def _log_softmax_sum(p0, p1):
    def body(a_ref, b_ref, o_ref):
        x = (a_ref[...] + b_ref[...])[:, :CLS]
        m = jnp.max(x, axis=1, keepdims=True)
        sh = x - m
        o_ref[...] = sh - jnp.log(jnp.sum(jnp.exp(sh), axis=1, keepdims=True))

    return pl.pallas_call(
        body,
        grid=(N // MBLK,),
        in_specs=[pl.BlockSpec((MBLK, 128), lambda i: (i, 0)),
                  pl.BlockSpec((MBLK, 128), lambda i: (i, 0))],
        out_specs=pl.BlockSpec((MBLK, CLS), lambda i: (i, 0)),
        out_shape=jax.ShapeDtypeStruct((N, CLS), jnp.float32),
    )(p0, p1)


def kernel(inputs, edge_index, W0, W1, W2, epoch):
    edges3d = edge_index.reshape(2, NCHUNK, K).transpose(1, 0, 2)
    w2p = jnp.pad(W2, ((0, 0), (0, 128 - CLS)))

    h0a, h0b = _gemm0(inputs, W0)
    a0a, a0b = _spmm128(h0a, h0b, edges3d)
    h1a, h1b = _gemm_relu(a0a, a0b, W1, HID)
    a1a, a1b = _spmm128(h1a, h1b, edges3d)
    h2 = _gemm_relu_wide(a1a, a1b, w2p)
    p0, p1 = _spmm_last(h2, edges3d)
    return _log_softmax_sum(p0, p1)
